# Initial kernel scaffold; baseline (speedup 1.0000x reference)
import jax
import jax.numpy as jnp
from jax.experimental import pallas as pl


def kernel(h, adj, W_lg, b_lg):
    raise NotImplementedError("write your pallas kernel here")


# speedup vs baseline: 1.6512x; 1.6512x over previous
import functools

import jax
import jax.numpy as jnp
from jax.experimental import pallas as pl

_N = 8192
_D = 64
_BM = 256


def _fused_block(adj_ref, h_ref, hblk_ref, wt_ref, b_ref, out_ref):
    adj = adj_ref[...]
    acc = jnp.dot(adj, h_ref[...], preferred_element_type=jnp.float32)
    rs = jnp.sum(adj, axis=1, keepdims=True)
    pooled = acc / rs
    cat = jnp.concatenate([hblk_ref[...], pooled], axis=1)
    out_ref[...] = (
        jnp.dot(cat, wt_ref[...], preferred_element_type=jnp.float32) + b_ref[...]
    )


@jax.jit
def kernel(h, adj, W_lg, b_lg):
    n, d = h.shape
    wt = W_lg.T
    b = b_lg.reshape(1, d)
    grid = (n // _BM,)
    return pl.pallas_call(
        _fused_block,
        grid=grid,
        in_specs=[
            pl.BlockSpec((_BM, n), lambda i: (i, 0)),
            pl.BlockSpec((n, d), lambda i: (0, 0)),
            pl.BlockSpec((_BM, d), lambda i: (i, 0)),
            pl.BlockSpec((2 * d, d), lambda i: (0, 0)),
            pl.BlockSpec((1, d), lambda i: (0, 0)),
        ],
        out_specs=pl.BlockSpec((_BM, d), lambda i: (i, 0)),
        out_shape=jax.ShapeDtypeStruct((n, d), jnp.float32),
    )(adj, h, h, wt, b)
